# Initial kernel scaffold; baseline (speedup 1.0000x reference)
import jax
import jax.numpy as jnp
from jax.experimental import pallas as pl


def kernel(x, gate_w, gate_b, expert_w, expert_b):
    raise NotImplementedError("write your pallas kernel here")


# speedup vs baseline: 24.4323x; 24.4323x over previous
import functools

import jax
import jax.numpy as jnp
from jax.experimental import pallas as pl
from jax.experimental.pallas import tpu as pltpu


def _moe_block(x_ref, gw_ref, gb_ref, ew_ref, eb_ref, out_ref, scale_ref):
    e = pl.program_id(1)
    n_e = pl.num_programs(1)
    x = x_ref[...]

    @pl.when(e == 0)
    def _gate():
        logits = jnp.dot(x, gw_ref[...].T, preferred_element_type=jnp.float32)
        logits = logits + gb_ref[...]
        m = jnp.max(logits, axis=-1, keepdims=True)
        top_gate = 1.0 / jnp.sum(jnp.exp(logits - m), axis=-1)
        n_exp = logits.shape[-1]
        iota = jax.lax.broadcasted_iota(jnp.int32, logits.shape, 1)
        amax = jnp.min(jnp.where(logits == m, iota, n_exp), axis=-1)
        scale_ref[:, 0] = n_exp * top_gate
        scale_ref[:, 1] = amax.astype(jnp.float32)

    y = jnp.dot(x, ew_ref[0].T, preferred_element_type=jnp.float32)
    y = y + eb_ref[0]
    sel = scale_ref[:, 1].astype(jnp.int32) == e
    w = jnp.where(sel, scale_ref[:, 0], 0.0)
    contrib = w[:, None] * y

    @pl.when(e == 0)
    def _init():
        out_ref[...] = contrib

    @pl.when(e > 0)
    def _acc():
        out_ref[...] = out_ref[...] + contrib


def kernel(x, gate_w, gate_b, expert_w, expert_b):
    B, S, H = x.shape
    E = gate_w.shape[0]
    N = B * S
    x2 = x.reshape(N, H)
    T = min(1024, N)
    C = N // T

    out = pl.pallas_call(
        _moe_block,
        grid=(C, E),
        in_specs=[
            pl.BlockSpec((T, H), lambda c, e: (c, 0)),
            pl.BlockSpec((E, H), lambda c, e: (0, 0)),
            pl.BlockSpec((E,), lambda c, e: (0,)),
            pl.BlockSpec((1, H, H), lambda c, e: (e, 0, 0)),
            pl.BlockSpec((1, 1, H), lambda c, e: (e, 0, 0)),
        ],
        out_specs=pl.BlockSpec((T, H), lambda c, e: (c, 0)),
        out_shape=jax.ShapeDtypeStruct((N, H), jnp.float32),
        scratch_shapes=[pltpu.VMEM((T, 2), jnp.float32)],
    )(x2, gate_w, gate_b, expert_w, expert_b.reshape(E, 1, H))
    return out.reshape(B, S, H)


# speedup vs baseline: 24.6348x; 1.0083x over previous
import functools

import jax
import jax.numpy as jnp
from jax.experimental import pallas as pl
from jax.experimental.pallas import tpu as pltpu


def _moe_block(x_ref, gw_ref, gb_ref, ew_ref, eb_ref, out_ref, scale_ref):
    e = pl.program_id(1)
    n_e = pl.num_programs(1)
    x = x_ref[...]

    @pl.when(e == 0)
    def _gate():
        logits = jnp.dot(x, gw_ref[...].T, preferred_element_type=jnp.float32)
        logits = logits + gb_ref[...]
        m = jnp.max(logits, axis=-1, keepdims=True)
        top_gate = 1.0 / jnp.sum(jnp.exp(logits - m), axis=-1)
        n_exp = logits.shape[-1]
        iota = jax.lax.broadcasted_iota(jnp.int32, logits.shape, 1)
        amax = jnp.min(jnp.where(logits == m, iota, n_exp), axis=-1)
        scale_ref[:, 0] = n_exp * top_gate
        scale_ref[:, 1] = amax.astype(jnp.float32)

    xb = x.astype(jnp.bfloat16)
    wb = ew_ref[0].astype(jnp.bfloat16)
    y = jnp.dot(xb, wb.T, preferred_element_type=jnp.float32)
    y = y + eb_ref[0]
    sel = scale_ref[:, 1].astype(jnp.int32) == e
    w = jnp.where(sel, scale_ref[:, 0], 0.0)
    contrib = w[:, None] * y

    @pl.when(e == 0)
    def _init():
        out_ref[...] = contrib

    @pl.when(e > 0)
    def _acc():
        out_ref[...] = out_ref[...] + contrib


def kernel(x, gate_w, gate_b, expert_w, expert_b):
    B, S, H = x.shape
    E = gate_w.shape[0]
    N = B * S
    x2 = x.reshape(N, H)
    T = min(2048, N)
    C = N // T

    out = pl.pallas_call(
        _moe_block,
        grid=(C, E),
        in_specs=[
            pl.BlockSpec((T, H), lambda c, e: (c, 0)),
            pl.BlockSpec((E, H), lambda c, e: (0, 0)),
            pl.BlockSpec((E,), lambda c, e: (0,)),
            pl.BlockSpec((1, H, H), lambda c, e: (e, 0, 0)),
            pl.BlockSpec((1, 1, H), lambda c, e: (e, 0, 0)),
        ],
        out_specs=pl.BlockSpec((T, H), lambda c, e: (c, 0)),
        out_shape=jax.ShapeDtypeStruct((N, H), jnp.float32),
        scratch_shapes=[pltpu.VMEM((T, 2), jnp.float32)],
    )(x2, gate_w, gate_b, expert_w, expert_b.reshape(E, 1, H))
    return out.reshape(B, S, H)
